# Initial kernel scaffold; baseline (speedup 1.0000x reference)
import jax
import jax.numpy as jnp
from jax.experimental import pallas as pl


def kernel(x, x_segment, token_table, segment_table, position_table):
    raise NotImplementedError("write your pallas kernel here")


# speedup vs baseline: 1.0353x; 1.0353x over previous
import functools

import jax
import jax.numpy as jnp
from jax import lax
from jax.experimental import pallas as pl
from jax.experimental.pallas import tpu as pltpu
from jax.experimental.pallas import tpu_sc as plsc

B = 1024
L = 200
D = 128
N = B * L
NW = 32
PER_W = N // NW
CHUNK = 128
NCHUNK = PER_W // CHUNK
NLANE = 16


def _emb_body(x_hbm, seg_hbm, tok_hbm, segtab_hbm, postab_hbm, out_hbm,
              tidx, sidx, tok_buf, pos_buf, seg_buf, sem):
    wid = lax.axis_index("s") * 2 + lax.axis_index("c")
    base0 = wid * PER_W

    pltpu.sync_copy(postab_hbm.at[pl.ds(0, L)], pos_buf)
    pltpu.sync_copy(segtab_hbm, seg_buf)

    lanes = jax.lax.iota(jnp.int32, NLANE)

    def chunk_body(ci, _):
        base = base0 + ci * CHUNK
        pltpu.sync_copy(x_hbm.at[pl.ds(base, CHUNK)], tidx)
        pltpu.sync_copy(seg_hbm.at[pl.ds(base, CHUNK)], sidx)
        pltpu.async_copy(tok_hbm.at[tidx], tok_buf, sem).wait()

        def group_body(g, _):
            rows = lanes + g * NLANE
            lvec = lax.rem(rows + base, jnp.int32(L))
            sv = sidx[pl.ds(g * NLANE, NLANE)]
            for c2 in range(D):
                col = jnp.full((NLANE,), c2, jnp.int32)
                t = plsc.load_gather(tok_buf, [rows, col])
                p = plsc.load_gather(pos_buf, [lvec, col])
                s = plsc.load_gather(seg_buf, [sv, col])
                plsc.store_scatter(tok_buf, [rows, col], t + p + s)
            return 0

        lax.fori_loop(0, CHUNK // NLANE, group_body, 0)
        pltpu.sync_copy(tok_buf, out_hbm.at[pl.ds(base, CHUNK)])
        return 0

    lax.fori_loop(0, NCHUNK, chunk_body, 0)


def kernel(x, x_segment, token_table, segment_table, position_table):
    mesh = plsc.VectorSubcoreMesh(core_axis_name="c", subcore_axis_name="s")
    run = functools.partial(
        pl.kernel,
        mesh=mesh,
        compiler_params=pltpu.CompilerParams(needs_layout_passes=False),
        out_type=jax.ShapeDtypeStruct((N, D), jnp.float32),
        scratch_types=[
            pltpu.VMEM((CHUNK,), jnp.int32),
            pltpu.VMEM((CHUNK,), jnp.int32),
            pltpu.VMEM((CHUNK, D), jnp.float32),
            pltpu.VMEM((L, D), jnp.float32),
            pltpu.VMEM((3, D), jnp.float32),
            pltpu.SemaphoreType.DMA,
        ],
    )(_emb_body)
    out = run(
        x.reshape(N),
        x_segment.reshape(N),
        token_table,
        segment_table,
        position_table,
    )
    return out.reshape(B, L, D)


# speedup vs baseline: 4.5333x; 4.3786x over previous
import functools

import jax
import jax.numpy as jnp
from jax import lax
from jax.experimental import pallas as pl
from jax.experimental.pallas import tpu as pltpu
from jax.experimental.pallas import tpu_sc as plsc

B = 1024
L = 200
D = 128
N = B * L
NW = 32
PER_W = N // NW
CHUNK = 128
NCHUNK = PER_W // CHUNK
NLANE = 16


def _emb_body(x_hbm, seg_hbm, tok_hbm, segtab_hbm, postab_hbm, out_hbm,
              tidx, sidx, tok_buf, ps3, seg_buf, sem):
    wid = lax.axis_index("s") * 2 + lax.axis_index("c")
    base0 = wid * PER_W

    pltpu.sync_copy(segtab_hbm, seg_buf)
    for s in range(3):
        pltpu.sync_copy(postab_hbm.at[pl.ds(0, L)],
                        ps3.at[pl.ds(s * L, L)])

    def ps3_body(i, _):
        for s in range(3):
            row = s * L + i
            for j in range(D // NLANE):
                sl = pl.ds(j * NLANE, NLANE)
                ps3[row, sl] = ps3[row, sl] + seg_buf[s, sl]
        return 0

    lax.fori_loop(0, L, ps3_body, 0)

    lanes = jax.lax.iota(jnp.int32, NLANE)

    def chunk_body(ci, _):
        base = base0 + ci * CHUNK
        pltpu.sync_copy(x_hbm.at[pl.ds(base, CHUNK)], tidx)
        pltpu.sync_copy(seg_hbm.at[pl.ds(base, CHUNK)], sidx)
        pltpu.async_copy(tok_hbm.at[tidx], tok_buf, sem).wait()

        def group_body(g, _):
            rows = lanes + g * NLANE
            lvec = lax.rem(rows + base, jnp.int32(L))
            sv = sidx[pl.ds(g * NLANE, NLANE)]
            psrow = sv * jnp.int32(L) + lvec
            for c in range(D):
                colv = lax.bitwise_and(lanes + c, jnp.int32(D - 1))
                t = plsc.load_gather(tok_buf, [rows, colv])
                p = plsc.load_gather(ps3, [psrow, colv])
                plsc.store_scatter(tok_buf, [rows, colv], t + p)
            return 0

        lax.fori_loop(0, CHUNK // NLANE, group_body, 0)
        pltpu.sync_copy(tok_buf, out_hbm.at[pl.ds(base, CHUNK)])
        return 0

    lax.fori_loop(0, NCHUNK, chunk_body, 0)


def kernel(x, x_segment, token_table, segment_table, position_table):
    mesh = plsc.VectorSubcoreMesh(core_axis_name="c", subcore_axis_name="s")
    run = functools.partial(
        pl.kernel,
        mesh=mesh,
        compiler_params=pltpu.CompilerParams(needs_layout_passes=False),
        out_type=jax.ShapeDtypeStruct((N, D), jnp.float32),
        scratch_types=[
            pltpu.VMEM((CHUNK,), jnp.int32),
            pltpu.VMEM((CHUNK,), jnp.int32),
            pltpu.VMEM((CHUNK, D), jnp.float32),
            pltpu.VMEM((3 * L, D), jnp.float32),
            pltpu.VMEM((3, D), jnp.float32),
            pltpu.SemaphoreType.DMA,
        ],
    )(_emb_body)
    out = run(
        x.reshape(N),
        x_segment.reshape(N),
        token_table,
        segment_table,
        position_table,
    )
    return out.reshape(B, L, D)


# speedup vs baseline: 4.5831x; 1.0110x over previous
import functools

import jax
import jax.numpy as jnp
from jax import lax
from jax.experimental import pallas as pl
from jax.experimental.pallas import tpu as pltpu
from jax.experimental.pallas import tpu_sc as plsc

B = 1024
L = 200
D = 128
N = B * L
NW = 32
PER_W = N // NW
CHUNK = 128
NCHUNK = PER_W // CHUNK
NLANE = 16


def _emb_body(x_hbm, seg_hbm, tok_hbm, segtab_hbm, postab_hbm, out_hbm,
              tidx, sidx, tok_buf, out_buf, ps3, seg_buf, sem):
    wid = lax.axis_index("s") * 2 + lax.axis_index("c")
    base0 = wid * PER_W

    pltpu.sync_copy(segtab_hbm, seg_buf)
    for s in range(3):
        pltpu.sync_copy(postab_hbm.at[pl.ds(0, L)],
                        ps3.at[pl.ds(s * L, L)])

    def ps3_body(i, _):
        for s in range(3):
            row = s * L + i
            for j in range(D // NLANE):
                sl = pl.ds(j * NLANE, NLANE)
                ps3[row, sl] = ps3[row, sl] + seg_buf[s, sl]
        return 0

    lax.fori_loop(0, L, ps3_body, 0)

    lanes = jax.lax.iota(jnp.int32, NLANE)

    def chunk_body(ci, _):
        base = base0 + ci * CHUNK
        pltpu.sync_copy(x_hbm.at[pl.ds(base, CHUNK)], tidx)
        pltpu.sync_copy(seg_hbm.at[pl.ds(base, CHUNK)], sidx)
        pltpu.async_copy(tok_hbm.at[tidx], tok_buf, sem).wait()

        @plsc.parallel_loop(0, CHUNK // NLANE)
        def group_body(g):
            rows = lanes + g * NLANE
            lvec = lax.rem(rows + base, jnp.int32(L))
            sv = sidx[pl.ds(g * NLANE, NLANE)]
            psrow = sv * jnp.int32(L) + lvec
            for c in range(D):
                colv = lax.bitwise_and(lanes + c, jnp.int32(D - 1))
                t = plsc.load_gather(tok_buf, [rows, colv])
                p = plsc.load_gather(ps3, [psrow, colv])
                plsc.store_scatter(out_buf, [rows, colv], t + p)

        pltpu.sync_copy(out_buf, out_hbm.at[pl.ds(base, CHUNK)])
        return 0

    lax.fori_loop(0, NCHUNK, chunk_body, 0)


def kernel(x, x_segment, token_table, segment_table, position_table):
    mesh = plsc.VectorSubcoreMesh(core_axis_name="c", subcore_axis_name="s")
    run = functools.partial(
        pl.kernel,
        mesh=mesh,
        compiler_params=pltpu.CompilerParams(needs_layout_passes=False),
        out_type=jax.ShapeDtypeStruct((N, D), jnp.float32),
        scratch_types=[
            pltpu.VMEM((CHUNK,), jnp.int32),
            pltpu.VMEM((CHUNK,), jnp.int32),
            pltpu.VMEM((CHUNK, D), jnp.float32),
            pltpu.VMEM((CHUNK, D), jnp.float32),
            pltpu.VMEM((3 * L, D), jnp.float32),
            pltpu.VMEM((3, D), jnp.float32),
            pltpu.SemaphoreType.DMA,
        ],
    )(_emb_body)
    out = run(
        x.reshape(N),
        x_segment.reshape(N),
        token_table,
        segment_table,
        position_table,
    )
    return out.reshape(B, L, D)
